# Initial kernel scaffold; baseline (speedup 1.0000x reference)
import jax
import jax.numpy as jnp
from jax.experimental import pallas as pl


def kernel(C, vals, row, col):
    raise NotImplementedError("write your pallas kernel here")


# speedup vs baseline: 3.2093x; 3.2093x over previous
import functools

import jax
import jax.numpy as jnp
from jax import lax
from jax.experimental import pallas as pl
from jax.experimental.pallas import tpu as pltpu
from jax.experimental.pallas import tpu_sc as plsc

N_NODES = 10000
N_EDGES = 320000
D_FEAT = 128
ALPHA = 0.5
HOP = 3

NC = 2
NS = 16
NW = NC * NS
EPW = N_EDGES // NW
K = 80
NCHUNK = EPW // K
ZCH = N_NODES // K


def _scatter_body(cf_hbm, vals_hbm, row_hbm, col_hbm, out_hbm,
                  colv, rowv, vrep, gbuf, acc, sem):
    c = lax.axis_index("c")
    s = lax.axis_index("s")
    wid = c * NS + s

    zero16 = jnp.zeros((16,), jnp.float32)

    def zero_row(i, _):
        for r8 in range(8):
            gbuf[i, pl.ds(r8 * 16, 16)] = zero16
        return 0
    lax.fori_loop(0, K, zero_row, 0)
    for u in range(8):
        j = s + u * NS
        @pl.when(j < ZCH)
        def _():
            pltpu.sync_copy(gbuf, acc.at[pl.ds(j * K, K)])
    plsc.subcore_barrier()

    def chunk_body(i, _):
        pltpu.sync_copy(col_hbm.at[wid, i], colv)
        pltpu.sync_copy(row_hbm.at[wid, i], rowv)
        pltpu.sync_copy(vals_hbm.at[wid, i], vrep)
        pltpu.async_copy(cf_hbm.at[colv], gbuf, sem).wait()

        def edge_body(j, _):
            v16 = vrep[j]
            for r8 in range(8):
                sl = pl.ds(r8 * 16, 16)
                gbuf[j, sl] = gbuf[j, sl] * v16
            return 0
        lax.fori_loop(0, K, edge_body, 0)

        pltpu.sync_copy(gbuf, acc.at[rowv], add=True)
        return 0
    lax.fori_loop(0, NCHUNK, chunk_body, 0)
    plsc.subcore_barrier()

    for u in range(8):
        j = s + u * NS
        @pl.when(j < ZCH)
        def _():
            pltpu.sync_copy(acc.at[pl.ds(j * K, K)],
                            out_hbm.at[c, pl.ds(j * K, K)])


_scatter = pl.kernel(
    _scatter_body,
    out_type=jax.ShapeDtypeStruct((NC, N_NODES, D_FEAT), jnp.float32),
    mesh=plsc.VectorSubcoreMesh(core_axis_name="c", subcore_axis_name="s"),
    scratch_types=[
        pltpu.VMEM((K,), jnp.int32),
        pltpu.VMEM((K,), jnp.int32),
        pltpu.VMEM((K, 16), jnp.float32),
        pltpu.VMEM((K, D_FEAT), jnp.float32),
        pltpu.VMEM_SHARED((N_NODES, D_FEAT), jnp.float32),
        pltpu.SemaphoreType.DMA,
    ],
)


def _combine_body(ac_ref, c_ref, o_ref):
    o_ref[...] = ALPHA * (ac_ref[0] + ac_ref[1]) + c_ref[...]


_BR = 400

_combine = pl.pallas_call(
    _combine_body,
    out_shape=jax.ShapeDtypeStruct((N_NODES, D_FEAT), jnp.float32),
    grid=(N_NODES // _BR,),
    in_specs=[
        pl.BlockSpec((NC, _BR, D_FEAT), lambda i: (0, i, 0)),
        pl.BlockSpec((_BR, D_FEAT), lambda i: (i, 0)),
    ],
    out_specs=pl.BlockSpec((_BR, D_FEAT), lambda i: (i, 0)),
)


def kernel(C, vals, row, col):
    vals_r = jnp.broadcast_to(
        vals.reshape(NW, NCHUNK, K, 1), (NW, NCHUNK, K, 16)
    )
    row_r = row.reshape(NW, NCHUNK, K)
    col_r = col.reshape(NW, NCHUNK, K)
    cf = C
    for _ in range(HOP):
        ac = _scatter(cf, vals_r, row_r, col_r)
        cf = _combine(ac, C)
    return cf


# speedup vs baseline: 8.0691x; 2.5143x over previous
import functools

import jax
import jax.numpy as jnp
from jax import lax
from jax.experimental import pallas as pl
from jax.experimental.pallas import tpu as pltpu
from jax.experimental.pallas import tpu_sc as plsc

N_NODES = 10000
N_EDGES = 320000
D_FEAT = 128
ALPHA = 0.5
HOP = 3

NC = 2
NS = 16
NW = NC * NS
EPW = N_EDGES // NW
K = 80
NCHUNK = EPW // K
ZCH = N_NODES // K
UNROLL = 6


def _scatter_body(cf_hbm, vals_hbm, meta_hbm, out_hbm,
                  meta0, meta1, meta2, vrep0, vrep1,
                  gbuf0, gbuf1, acc,
                  semm0, semm1, semm2, semv0, semv1,
                  semg0, semg1, sems0, sems1):
    metas = (meta0, meta1, meta2)
    vreps = (vrep0, vrep1)
    gbufs = (gbuf0, gbuf1)
    semms = (semm0, semm1, semm2)
    semvs = (semv0, semv1)
    semgs = (semg0, semg1)
    semss = (sems0, sems1)

    c = lax.axis_index("c")
    s = lax.axis_index("s")
    wid = c * NS + s

    def fire_meta(i, t):
        pltpu.async_copy(meta_hbm.at[wid, i], metas[t], semms[t])

    def wait_meta(t):
        pltpu.make_async_copy(meta_hbm.at[wid, 0], metas[t], semms[t]).wait()

    def fire_vrep(i, b):
        pltpu.async_copy(vals_hbm.at[wid, i], vreps[b], semvs[b])

    def wait_vrep(b):
        pltpu.make_async_copy(vals_hbm.at[wid, 0], vreps[b], semvs[b]).wait()

    def fire_gather(b, t):
        pltpu.async_copy(cf_hbm.at[metas[t].at[0]], gbufs[b], semgs[b])

    def wait_gather(b, t):
        pltpu.make_async_copy(
            cf_hbm.at[metas[t].at[0]], gbufs[b], semgs[b]).wait()

    def fire_scatter(b, t):
        pltpu.async_copy(gbufs[b], acc.at[metas[t].at[1]], semss[b], add=True)

    def wait_scatter(b, t):
        pltpu.make_async_copy(gbufs[b], acc.at[metas[t].at[1]], semss[b]).wait()

    zero16 = jnp.zeros((16,), jnp.float32)

    def zero_row(i, _):
        for r8 in range(8):
            gbuf0[i, pl.ds(r8 * 16, 16)] = zero16
        return 0
    lax.fori_loop(0, K, zero_row, 0)
    for u in range(8):
        j = s + u * NS
        @pl.when(j < ZCH)
        def _():
            pltpu.sync_copy(gbuf0, acc.at[pl.ds(j * K, K)])
    plsc.subcore_barrier()

    fire_meta(0, 0)
    fire_meta(1, 1)
    fire_vrep(0, 0)
    fire_vrep(1, 1)
    wait_meta(0)
    fire_gather(0, 0)

    def sub(i, b, t):
        bn = 1 - b
        wait_gather(b, t)

        @pl.when(i + 1 < NCHUNK)
        def _():
            @pl.when(i >= 1)
            def _():
                wait_scatter(bn, (t + 2) % 3)
            wait_meta((t + 1) % 3)

            @pl.when(i + 2 < NCHUNK)
            def _():
                fire_meta(i + 2, (t + 2) % 3)
            fire_gather(bn, (t + 1) % 3)

        wait_vrep(b)

        @plsc.parallel_loop(0, K, step=1, unroll=4)
        def _(j):
            v16 = vreps[b][j]
            for r8 in range(8):
                sl = pl.ds(r8 * 16, 16)
                gbufs[b][j, sl] = gbufs[b][j, sl] * v16

        fire_scatter(b, t)

        @pl.when(i + 2 < NCHUNK)
        def _():
            fire_vrep(i + 2, b)

    def outer(io, _):
        for p in range(UNROLL):
            i = io * UNROLL + p
            @pl.when(i < NCHUNK)
            def _():
                sub(i, p % 2, p % 3)
        return 0
    lax.fori_loop(0, (NCHUNK + UNROLL - 1) // UNROLL, outer, 0)

    wait_scatter((NCHUNK - 2) % 2, (NCHUNK - 2) % 3)
    wait_scatter((NCHUNK - 1) % 2, (NCHUNK - 1) % 3)
    plsc.subcore_barrier()

    for u in range(8):
        j = s + u * NS
        @pl.when(j < ZCH)
        def _():
            pltpu.sync_copy(acc.at[pl.ds(j * K, K)],
                            out_hbm.at[c, pl.ds(j * K, K)])


_scatter = pl.kernel(
    _scatter_body,
    out_type=jax.ShapeDtypeStruct((NC, N_NODES, D_FEAT), jnp.float32),
    mesh=plsc.VectorSubcoreMesh(core_axis_name="c", subcore_axis_name="s"),
    scratch_types=[
        pltpu.VMEM((2, K), jnp.int32),
        pltpu.VMEM((2, K), jnp.int32),
        pltpu.VMEM((2, K), jnp.int32),
        pltpu.VMEM((K, 16), jnp.float32),
        pltpu.VMEM((K, 16), jnp.float32),
        pltpu.VMEM((K, D_FEAT), jnp.float32),
        pltpu.VMEM((K, D_FEAT), jnp.float32),
        pltpu.VMEM_SHARED((N_NODES, D_FEAT), jnp.float32),
        pltpu.SemaphoreType.DMA,
        pltpu.SemaphoreType.DMA,
        pltpu.SemaphoreType.DMA,
        pltpu.SemaphoreType.DMA,
        pltpu.SemaphoreType.DMA,
        pltpu.SemaphoreType.DMA,
        pltpu.SemaphoreType.DMA,
        pltpu.SemaphoreType.DMA,
        pltpu.SemaphoreType.DMA,
    ],
)


def _combine_body(ac_ref, c_ref, o_ref):
    o_ref[...] = ALPHA * (ac_ref[0] + ac_ref[1]) + c_ref[...]


_BR = 400

_combine = pl.pallas_call(
    _combine_body,
    out_shape=jax.ShapeDtypeStruct((N_NODES, D_FEAT), jnp.float32),
    grid=(N_NODES // _BR,),
    in_specs=[
        pl.BlockSpec((NC, _BR, D_FEAT), lambda i: (0, i, 0)),
        pl.BlockSpec((_BR, D_FEAT), lambda i: (i, 0)),
    ],
    out_specs=pl.BlockSpec((_BR, D_FEAT), lambda i: (i, 0)),
)


def kernel(C, vals, row, col):
    vals_r = jnp.broadcast_to(
        vals.reshape(NW, NCHUNK, K, 1), (NW, NCHUNK, K, 16)
    )
    meta = jnp.stack(
        [col.reshape(NW, NCHUNK, K), row.reshape(NW, NCHUNK, K)], axis=2
    )
    cf = C
    for _ in range(HOP):
        ac = _scatter(cf, vals_r, meta)
        cf = _combine(ac, C)
    return cf
